# baseline (speedup 1.0000x reference)
import functools

import jax
import jax.numpy as jnp
from jax.experimental import pallas as pl
from jax.experimental.pallas import tpu as pltpu


def _gru2_fc_kernel(x_ref,
                    wih0_ref, whh0_ref, bx0_ref, bhn0_ref,
                    w1_ref, b1_ref,
                    fcw_ref, fcb_ref,
                    out_ref, gx_scr, *, T, B, H):
    f32 = jnp.float32
    bf16 = jnp.bfloat16
    half = bf16(0.5)

    gx_scr[...] = jnp.dot(x_ref[...], wih0_ref[...],
                          preferred_element_type=f32).astype(bf16)

    whh0 = whh0_ref[...]
    w1 = w1_ref[...]
    bx0 = bx0_ref[...].astype(bf16)
    bhn0 = bhn0_ref[...].astype(bf16)
    b1 = b1_ref[...]

    def step0(g, gh, h):
        tr = jnp.tanh(g[:, 0 * H:1 * H] + gh[:, 0 * H:1 * H])
        tz = jnp.tanh(g[:, 1 * H:2 * H] + gh[:, 1 * H:2 * H])
        ghn = gh[:, 2 * H:3 * H] + bhn0
        n = jnp.tanh(g[:, 2 * H:3 * H] + ghn + tr * ghn)
        return half * ((h + n) + tz * (h - n))

    def step1(a, h):
        tr = jnp.tanh(a[:, 0 * H:1 * H])
        tz = jnp.tanh(a[:, 1 * H:2 * H])
        n = jnp.tanh(a[:, 2 * H:3 * H] + tr * a[:, 3 * H:4 * H])
        return half * ((h + n) + tz * (h - n))

    h0 = jnp.zeros((B, H), bf16)
    h1 = jnp.zeros((B, H), bf16)
    gh0 = jnp.zeros((B, 3 * H), bf16)
    a1 = None
    for t in range(T):
        row = pl.multiple_of(t * B, B)
        g0 = gx_scr[pl.ds(row, B), :] + bx0
        h0_new = step0(g0, gh0, h0)
        if a1 is not None:
            h1 = step1(a1, h1)
        if t < T - 1:
            gh0 = jnp.dot(h0_new, whh0,
                          preferred_element_type=f32).astype(bf16)
        a1 = (jnp.dot(jnp.concatenate([h0_new, h1], axis=1), w1,
                      preferred_element_type=f32) + b1).astype(bf16)
        h0 = h0_new
    h1 = step1(a1, h1)

    out_ref[...] = (jnp.dot(h1, fcw_ref[...], preferred_element_type=f32)
                    + fcb_ref[...]).astype(out_ref.dtype)


def _prep_layer0(w_ih, w_hh, b_ih, b_hh, H):
    bf16 = jnp.bfloat16
    scale = jnp.concatenate([jnp.full((2 * H,), 0.5, jnp.float32),
                             jnp.ones((H,), jnp.float32)])
    wih_t = (w_ih.T * scale[None, :]).astype(bf16)
    whh_t = (0.5 * w_hh.T).astype(bf16)
    bx = (scale * (b_ih + jnp.concatenate(
        [b_hh[:2 * H], jnp.zeros((H,), jnp.float32)]))).reshape(1, 3 * H)
    bhn = (0.5 * b_hh[2 * H:]).reshape(1, H)
    return wih_t, whh_t, bx, bhn


def _prep_layer1(w_ih, w_hh, b_ih, b_hh, H):
    bf16 = jnp.bfloat16
    wr_i, wz_i, wn_i = (w_ih[:H].T, w_ih[H:2 * H].T, w_ih[2 * H:].T)
    wr_h, wz_h, wn_h = (w_hh[:H].T, w_hh[H:2 * H].T, w_hh[2 * H:].T)
    zero = jnp.zeros_like(wn_i)
    top = jnp.concatenate([0.5 * wr_i, 0.5 * wz_i, wn_i, zero], axis=1)
    bot = jnp.concatenate([0.5 * wr_h, 0.5 * wz_h, 0.5 * wn_h, 0.5 * wn_h],
                          axis=1)
    w1 = jnp.concatenate([top, bot], axis=0).astype(bf16)
    b1 = jnp.concatenate([
        0.5 * (b_ih[:H] + b_hh[:H]),
        0.5 * (b_ih[H:2 * H] + b_hh[H:2 * H]),
        b_ih[2 * H:] + 0.5 * b_hh[2 * H:],
        0.5 * b_hh[2 * H:]]).reshape(1, 4 * H)
    return w1, b1


def kernel(w_ih_0, w_hh_0, b_ih_0, b_hh_0,
           w_ih_1, w_hh_1, b_ih_1, b_hh_1,
           fc_w, fc_b, x):
    B, T, D = x.shape
    H = w_hh_0.shape[1]
    C = fc_w.shape[0]
    bf16 = jnp.bfloat16

    wih0, whh0, bx0, bhn0 = _prep_layer0(w_ih_0, w_hh_0, b_ih_0, b_hh_0, H)
    w1, b1 = _prep_layer1(w_ih_1, w_hh_1, b_ih_1, b_hh_1, H)
    fcw = fc_w.T.astype(bf16)
    fcb = fc_b.reshape(1, C)

    x_flat = jnp.transpose(x, (1, 0, 2)).astype(bf16).reshape(T * B, D)

    operands = [x_flat, wih0, whh0, bx0, bhn0, w1, b1, fcw, fcb]
    in_specs = [pl.BlockSpec(a.shape, lambda i, nd=a.ndim: (0,) * nd)
                for a in operands]

    out = pl.pallas_call(
        functools.partial(_gru2_fc_kernel, T=T, B=B, H=H),
        out_shape=jax.ShapeDtypeStruct((B, C), jnp.float32),
        grid=(1,),
        in_specs=in_specs,
        out_specs=pl.BlockSpec((B, C), lambda i: (0, 0)),
        scratch_shapes=[
            pltpu.VMEM((T * B, 3 * H), bf16),
        ],
        compiler_params=pltpu.CompilerParams(
            dimension_semantics=("arbitrary",)),
    )(*operands)
    return out
